# baseline (speedup 1.0000x reference)
import functools

import jax
import jax.numpy as jnp
from jax.experimental import pallas as pl

N_NODES = 10000
ROW_BLK = 1000


def _leaky(x, slope):
    return jnp.where(x >= 0, x, slope * x)


def _ln_mm_body(x_ref, g_ref, b_ref, w_ref, wb_ref, o_ref):
    x = x_ref[...]
    mu = jnp.mean(x, axis=-1, keepdims=True)
    var = jnp.mean((x - mu) ** 2, axis=-1, keepdims=True)
    xn = (x - mu) * jax.lax.rsqrt(var + 1e-5) * g_ref[...] + b_ref[...]
    o_ref[...] = jnp.dot(xn, w_ref[...], preferred_element_type=jnp.float32) + wb_ref[...]


def _ln_mm(x, g, b, w, wb):
    n, f = x.shape
    fo = w.shape[1]
    grid = (n // ROW_BLK,)
    return pl.pallas_call(
        _ln_mm_body,
        grid=grid,
        in_specs=[
            pl.BlockSpec((ROW_BLK, f), lambda i: (i, 0)),
            pl.BlockSpec((f,), lambda i: (0,)),
            pl.BlockSpec((f,), lambda i: (0,)),
            pl.BlockSpec((f, fo), lambda i: (0, 0)),
            pl.BlockSpec((fo,), lambda i: (0,)),
        ],
        out_specs=pl.BlockSpec((ROW_BLK, fo), lambda i: (i, 0)),
        out_shape=jax.ShapeDtypeStruct((n, fo), jnp.float32),
    )(x, g, b, w, wb)


def _proj_body(h_ref, w_ref, asrc_ref, adst_ref, xl_ref, as_ref, ad_ref):
    h = h_ref[...]
    xl = jnp.dot(h, w_ref[...], preferred_element_type=jnp.float32)
    xl_ref[...] = xl
    as_ref[...] = (xl @ asrc_ref[...])[:, None]
    ad_ref[...] = (xl @ adst_ref[...])[:, None]


def _proj(h, w, att_src, att_dst):
    n, f = h.shape
    fo = w.shape[1]
    grid = (n // ROW_BLK,)
    return pl.pallas_call(
        _proj_body,
        grid=grid,
        in_specs=[
            pl.BlockSpec((ROW_BLK, f), lambda i: (i, 0)),
            pl.BlockSpec((f, fo), lambda i: (0, 0)),
            pl.BlockSpec((fo,), lambda i: (0,)),
            pl.BlockSpec((fo,), lambda i: (0,)),
        ],
        out_specs=[
            pl.BlockSpec((ROW_BLK, fo), lambda i: (i, 0)),
            pl.BlockSpec((ROW_BLK, 1), lambda i: (i, 0)),
            pl.BlockSpec((ROW_BLK, 1), lambda i: (i, 0)),
        ],
        out_shape=[
            jax.ShapeDtypeStruct((n, fo), jnp.float32),
            jax.ShapeDtypeStruct((n, 1), jnp.float32),
            jax.ShapeDtypeStruct((n, 1), jnp.float32),
        ],
    )(h, w, att_src, att_dst)


def _edge_phase(xl, a_src, a_dst, src, dst, bias):
    n = xl.shape[0]
    alpha = _leaky(a_src[src] + a_dst[dst], 0.2)
    amax = jax.ops.segment_max(alpha, dst, num_segments=n)
    ex = jnp.exp(alpha - amax[dst])
    denom = jax.ops.segment_sum(ex, dst, num_segments=n)
    coef = ex / (denom[dst] + 1e-16)
    msg = xl[src] * coef[:, None]
    out = jax.ops.segment_sum(msg, dst, num_segments=n)
    return out + bias


def kernel(x, edge_index, edge_attr, fc1_ln_g, fc1_ln_b, fc1_W, fc1_b, W1, att_src1, att_dst1, b1, W2, att_src2, att_dst2, b2, fc2_ln_g, fc2_ln_b, fc2_W, fc2_b):
    n = x.shape[0]
    loop = jnp.arange(n, dtype=edge_index.dtype)
    src = jnp.concatenate([edge_index[0], loop])
    dst = jnp.concatenate([edge_index[1], loop])

    h = _ln_mm(x, fc1_ln_g, fc1_ln_b, fc1_W, fc1_b)

    xl1, as1, ad1 = _proj(h, W1, att_src1, att_dst1)
    h = _leaky(_edge_phase(xl1, as1[:, 0], ad1[:, 0], src, dst, b1), 0.01)

    xl2, as2, ad2 = _proj(h, W2, att_src2, att_dst2)
    h = _leaky(_edge_phase(xl2, as2[:, 0], ad2[:, 0], src, dst, b2), 0.01)

    out = _ln_mm(h, fc2_ln_g, fc2_ln_b, fc2_W, fc2_b)
    return out


# speedup vs baseline: 19.5354x; 19.5354x over previous
import functools

import jax
import jax.numpy as jnp
from jax import lax
from jax.experimental import pallas as pl
from jax.experimental.pallas import tpu as pltpu
from jax.experimental.pallas import tpu_sc as plsc

N_NODES = 10000
N_PAD = 10240
NC, NS, L = 2, 16, 16
SEG = N_PAD // NS
E_PAD = 655360
EB1 = 512
B2 = 64
ROW_BLK = 1024


def _leaky(x, slope):
    return jnp.where(x >= 0, x, slope * x)


def _stage_a_body(x_ref, g_ref, b_ref, w1_ref, b1_ref, w2_ref, s_ref, d_ref,
                  xl_ref, as_ref, ad_ref, am_ref):
    x = x_ref[...]
    mu = jnp.mean(x, axis=-1, keepdims=True)
    var = jnp.mean((x - mu) ** 2, axis=-1, keepdims=True)
    xn = (x - mu) * jax.lax.rsqrt(var + 1e-5) * g_ref[...] + b_ref[...]
    h = jnp.dot(xn, w1_ref[...], preferred_element_type=jnp.float32) + b1_ref[...]
    xl = jnp.dot(h, w2_ref[...], preferred_element_type=jnp.float32)
    xl_ref[...] = xl
    asv = xl @ s_ref[...]
    as_ref[...] = asv[:, None]
    ad_ref[...] = (xl @ d_ref[...])[:, None]

    @pl.when(pl.program_id(0) == 0)
    def _():
        am_ref[...] = jnp.full((L,), -3e38, jnp.float32)
    am_ref[...] = jnp.maximum(am_ref[...], jnp.full((L,), jnp.max(asv)))


def _stage_a(x, g, b, w1, b1, w2, att_s, att_d):
    n, f = x.shape
    fo = w2.shape[1]
    return pl.pallas_call(
        _stage_a_body,
        grid=(n // ROW_BLK,),
        in_specs=[
            pl.BlockSpec((ROW_BLK, f), lambda i: (i, 0)),
            pl.BlockSpec((f,), lambda i: (0,)),
            pl.BlockSpec((f,), lambda i: (0,)),
            pl.BlockSpec((f, f), lambda i: (0, 0)),
            pl.BlockSpec((f,), lambda i: (0,)),
            pl.BlockSpec((f, fo), lambda i: (0, 0)),
            pl.BlockSpec((fo,), lambda i: (0,)),
            pl.BlockSpec((fo,), lambda i: (0,)),
        ],
        out_specs=[
            pl.BlockSpec((ROW_BLK, fo), lambda i: (i, 0)),
            pl.BlockSpec((ROW_BLK, 1), lambda i: (i, 0)),
            pl.BlockSpec((ROW_BLK, 1), lambda i: (i, 0)),
            pl.BlockSpec((L,), lambda i: (0,)),
        ],
        out_shape=[
            jax.ShapeDtypeStruct((n, fo), jnp.float32),
            jax.ShapeDtypeStruct((n, 1), jnp.float32),
            jax.ShapeDtypeStruct((n, 1), jnp.float32),
            jax.ShapeDtypeStruct((L,), jnp.float32),
        ],
    )(x, g, b, w1, b1, w2, att_s, att_d)


def _stage_b_body(p_ref, bias_ref, w_ref, s_ref, d_ref, xl_ref, as_ref, ad_ref,
                  am_ref):
    h = _leaky(p_ref[0] + p_ref[1] + bias_ref[...], 0.01)
    xl = jnp.dot(h, w_ref[...], preferred_element_type=jnp.float32)
    xl_ref[...] = xl
    asv = xl @ s_ref[...]
    as_ref[...] = asv[:, None]
    ad_ref[...] = (xl @ d_ref[...])[:, None]

    @pl.when(pl.program_id(0) == 0)
    def _():
        am_ref[...] = jnp.full((L,), -3e38, jnp.float32)
    am_ref[...] = jnp.maximum(am_ref[...], jnp.full((L,), jnp.max(asv)))


def _stage_b(p, bias, w, att_s, att_d):
    _, n, f = p.shape
    fo = w.shape[1]
    return pl.pallas_call(
        _stage_b_body,
        grid=(n // ROW_BLK,),
        in_specs=[
            pl.BlockSpec((2, ROW_BLK, f), lambda i: (0, i, 0)),
            pl.BlockSpec((f,), lambda i: (0,)),
            pl.BlockSpec((f, fo), lambda i: (0, 0)),
            pl.BlockSpec((fo,), lambda i: (0,)),
            pl.BlockSpec((fo,), lambda i: (0,)),
        ],
        out_specs=[
            pl.BlockSpec((ROW_BLK, fo), lambda i: (i, 0)),
            pl.BlockSpec((ROW_BLK, 1), lambda i: (i, 0)),
            pl.BlockSpec((ROW_BLK, 1), lambda i: (i, 0)),
            pl.BlockSpec((L,), lambda i: (0,)),
        ],
        out_shape=[
            jax.ShapeDtypeStruct((n, fo), jnp.float32),
            jax.ShapeDtypeStruct((n, 1), jnp.float32),
            jax.ShapeDtypeStruct((n, 1), jnp.float32),
            jax.ShapeDtypeStruct((L,), jnp.float32),
        ],
    )(p, bias, w, att_s, att_d)


def _stage_c_body(p_ref, bias_ref, g_ref, b_ref, w_ref, wb_ref, o_ref):
    nf = g_ref.shape[0]
    h = _leaky(p_ref[0] + p_ref[1] + bias_ref[...], 0.01)[:, :nf]
    mu = jnp.mean(h, axis=-1, keepdims=True)
    var = jnp.mean((h - mu) ** 2, axis=-1, keepdims=True)
    hn = (h - mu) * jax.lax.rsqrt(var + 1e-5) * g_ref[...] + b_ref[...]
    o_ref[...] = jnp.dot(hn, w_ref[...], preferred_element_type=jnp.float32) + wb_ref[...]


def _stage_c(p, bias, g, b, w, wb):
    _, n, f = p.shape
    fn = w.shape[0]
    fo = w.shape[1]
    return pl.pallas_call(
        _stage_c_body,
        grid=(n // ROW_BLK,),
        in_specs=[
            pl.BlockSpec((2, ROW_BLK, f), lambda i: (0, i, 0)),
            pl.BlockSpec((f,), lambda i: (0,)),
            pl.BlockSpec((fn,), lambda i: (0,)),
            pl.BlockSpec((fn,), lambda i: (0,)),
            pl.BlockSpec((fn, fo), lambda i: (0, 0)),
            pl.BlockSpec((fo,), lambda i: (0,)),
        ],
        out_specs=pl.BlockSpec((ROW_BLK, fo), lambda i: (i, 0)),
        out_shape=jax.ShapeDtypeStruct((n, fo), jnp.float32),
    )(p, bias, g, b, w, wb)


def _edge_ex(sbuf, dbuf, asrc_v, adst_v, amax_v, g):
    s = sbuf[pl.ds(g * L, L)]
    d = dbuf[pl.ds(g * L, L)]
    asv = plsc.load_gather(asrc_v, [s])
    adv = plsc.load_gather(adst_v, [d])
    al = _leaky(asv + adv, 0.2)
    mm = _leaky(amax_v + adv, 0.2)
    return d, jnp.exp(al - mm)


def _denom_body(src_hbm, dst_hbm, asrc_hbm, adst_hbm, amax_hbm, denom_hbm,
                asrc_v, adst_v, dpriv_v, sbuf, dbuf, cbuf, amax_b, slab_sh):
    cid = lax.axis_index("c")
    sid = lax.axis_index("s")
    pltpu.sync_copy(asrc_hbm, asrc_v)
    pltpu.sync_copy(adst_hbm, adst_v)
    pltpu.sync_copy(amax_hbm, amax_b)
    amax_v = amax_b[...]

    zero_v = jnp.zeros((L,), jnp.float32)

    def z(i, _):
        dpriv_v[pl.ds(i * L, L)] = zero_v
        return 0
    lax.fori_loop(0, N_PAD // L, z, 0)

    base_e = sid * (E_PAD // NS)

    def batch(b, _):
        off = base_e + b * EB1
        pltpu.sync_copy(src_hbm.at[pl.ds(off, EB1)], sbuf)
        pltpu.sync_copy(dst_hbm.at[pl.ds(off, EB1)], dbuf)

        def grp(g, _):
            d, ex = _edge_ex(sbuf, dbuf, asrc_v, adst_v, amax_v, g)
            plsc.addupdate_scatter(dpriv_v, [d], ex)
            return 0
        lax.fori_loop(0, EB1 // L, grp, 0)
        return 0
    lax.fori_loop(0, E_PAD // NS // EB1, batch, 0)

    pltpu.sync_copy(dpriv_v, slab_sh.at[sid])
    plsc.subcore_barrier()
    pltpu.sync_copy(slab_sh.at[:, pl.ds(sid * SEG, SEG)], cbuf)

    def red(j, _):
        acc = cbuf[0, pl.ds(j * L, L)]
        for r in range(1, NS):
            acc = acc + cbuf[r, pl.ds(j * L, L)]
        dpriv_v[pl.ds(j * L, L)] = acc
        return 0
    lax.fori_loop(0, SEG // L, red, 0)

    @pl.when(cid == 0)
    def _():
        pltpu.sync_copy(dpriv_v.at[pl.ds(0, SEG)], denom_hbm.at[pl.ds(sid * SEG, SEG)])


def _denom_kernel(src, dst, asrc, adst, amax16):
    mesh = plsc.VectorSubcoreMesh(core_axis_name="c", subcore_axis_name="s")
    return pl.kernel(
        _denom_body,
        out_type=jax.ShapeDtypeStruct((N_PAD,), jnp.float32),
        mesh=mesh,
        compiler_params=pltpu.CompilerParams(needs_layout_passes=False),
        scratch_types=[
            pltpu.VMEM((N_PAD,), jnp.float32),
            pltpu.VMEM((N_PAD,), jnp.float32),
            pltpu.VMEM((N_PAD,), jnp.float32),
            pltpu.VMEM((EB1,), jnp.int32),
            pltpu.VMEM((EB1,), jnp.int32),
            pltpu.VMEM((NS, SEG), jnp.float32),
            pltpu.VMEM((L,), jnp.float32),
            pltpu.VMEM_SHARED((NS, N_PAD), jnp.float32),
        ],
    )(src, dst, asrc, adst, amax16)


def _msg_body(nf, src_hbm, dst_hbm, asrc_hbm, adst_hbm, amax_hbm, denom_hbm,
              xl_hbm, out_hbm, asrc_v, adst_v, den_v, sbuf, dbuf, coef_v,
              rows_v, zrow_v, amax_b, acc_sh, sem):
    cid = lax.axis_index("c")
    sid = lax.axis_index("s")
    pltpu.sync_copy(asrc_hbm, asrc_v)
    pltpu.sync_copy(adst_hbm, adst_v)
    pltpu.sync_copy(denom_hbm, den_v)
    pltpu.sync_copy(amax_hbm, amax_b)
    amax_v = amax_b[...]

    zero_v = jnp.zeros((L,), jnp.float32)
    nvec = nf // L

    def zr(i, _):
        r = i // nvec
        c = i % nvec
        zrow_v[r, pl.ds(c * L, L)] = zero_v
        return 0
    lax.fori_loop(0, B2 * nvec, zr, 0)

    def zacc(i, _):
        pltpu.sync_copy(zrow_v, acc_sh.at[pl.ds(sid * SEG + i * B2, B2)])
        return 0
    lax.fori_loop(0, SEG // B2, zacc, 0)
    plsc.subcore_barrier()

    base_e = cid * (E_PAD // NC) + sid * (E_PAD // NC // NS)

    def batch(b, _):
        off = base_e + b * B2
        pltpu.sync_copy(src_hbm.at[pl.ds(off, B2)], sbuf)
        pltpu.sync_copy(dst_hbm.at[pl.ds(off, B2)], dbuf)
        pltpu.async_copy(xl_hbm.at[sbuf], rows_v, sem).wait()

        def grp(g, _):
            d, ex = _edge_ex(sbuf, dbuf, asrc_v, adst_v, amax_v, g)
            dnv = plsc.load_gather(den_v, [d])
            coef_v[pl.ds(g * L, L)] = ex / (dnv + 1e-30)
            return 0
        lax.fori_loop(0, B2 // L, grp, 0)

        def rowmul(g, _):
            cvec = coef_v[pl.ds(g * L, L)]
            for rr in range(L):
                r = g * L + rr
                cv = jnp.full((L,), cvec[rr], jnp.float32)
                for j in range(nvec):
                    rows_v[r, pl.ds(j * L, L)] = rows_v[r, pl.ds(j * L, L)] * cv
            return 0
        lax.fori_loop(0, B2 // L, rowmul, 0)

        pltpu.sync_copy(rows_v, acc_sh.at[dbuf], add=True)
        return 0
    lax.fori_loop(0, E_PAD // NC // NS // B2, batch, 0)

    plsc.subcore_barrier()
    pltpu.sync_copy(acc_sh.at[pl.ds(sid * SEG, SEG)],
                    out_hbm.at[cid, pl.ds(sid * SEG, SEG)])


def _msg_kernel(src, dst, asrc, adst, amax16, denom, xl):
    nf = xl.shape[1]
    mesh = plsc.VectorSubcoreMesh(core_axis_name="c", subcore_axis_name="s")
    return pl.kernel(
        functools.partial(_msg_body, nf),
        out_type=jax.ShapeDtypeStruct((NC, N_PAD, nf), jnp.float32),
        mesh=mesh,
        compiler_params=pltpu.CompilerParams(needs_layout_passes=False),
        scratch_types=[
            pltpu.VMEM((N_PAD,), jnp.float32),
            pltpu.VMEM((N_PAD,), jnp.float32),
            pltpu.VMEM((N_PAD,), jnp.float32),
            pltpu.VMEM((B2,), jnp.int32),
            pltpu.VMEM((B2,), jnp.int32),
            pltpu.VMEM((B2,), jnp.float32),
            pltpu.VMEM((B2, nf), jnp.float32),
            pltpu.VMEM((B2, nf), jnp.float32),
            pltpu.VMEM((L,), jnp.float32),
            pltpu.VMEM_SHARED((N_PAD, nf), jnp.float32),
            pltpu.SemaphoreType.DMA,
        ],
    )(src, dst, asrc, adst, amax16, denom, xl)


def _gat_edge(src, dst, xl, asrc, adst, amax16):
    denom = _denom_kernel(src, dst, asrc, adst, amax16)
    return _msg_kernel(src, dst, asrc, adst, amax16, denom, xl)


def kernel(x, edge_index, edge_attr, fc1_ln_g, fc1_ln_b, fc1_W, fc1_b, W1,
           att_src1, att_dst1, b1, W2, att_src2, att_dst2, b2, fc2_ln_g,
           fc2_ln_b, fc2_W, fc2_b):
    n = x.shape[0]
    loop = jnp.arange(n, dtype=jnp.int32)
    pad_e = jnp.full((E_PAD - edge_index.shape[1] - n,), N_NODES, jnp.int32)
    src = jnp.concatenate([edge_index[0], loop, pad_e])
    dst = jnp.concatenate([edge_index[1], loop, pad_e])

    x_pad = jnp.zeros((N_PAD, x.shape[1]), jnp.float32).at[:n].set(x)

    xl1, as1, ad1, am1 = _stage_a(x_pad, fc1_ln_g, fc1_ln_b, fc1_W, fc1_b, W1,
                                  att_src1, att_dst1)
    p1 = _gat_edge(src, dst, xl1, as1[:, 0], ad1[:, 0], am1)

    fo2 = W2.shape[1]
    fpad = 128 - fo2
    W2p = jnp.pad(W2, ((0, 0), (0, fpad)))
    att_src2p = jnp.pad(att_src2, (0, fpad))
    att_dst2p = jnp.pad(att_dst2, (0, fpad))
    b2p = jnp.pad(b2, (0, fpad))

    xl2, as2, ad2, am2 = _stage_b(p1, b1, W2p, att_src2p, att_dst2p)
    p2 = _gat_edge(src, dst, xl2, as2[:, 0], ad2[:, 0], am2)

    out = _stage_c(p2, b2p, fc2_ln_g, fc2_ln_b, fc2_W, fc2_b)
    return out[:n]
